# baseline (speedup 1.0000x reference)
import functools

import jax
import jax.numpy as jnp
from jax.experimental import pallas as pl
from jax.experimental.pallas import tpu as pltpu

_TWO_LOG2E = 2.0 * 1.4426950408889634


def _adjacency_block(refs_blk_ref, refs_ref, lsq_ref, srcs_ref, out_ref, *,
                     block_n):
    i = pl.program_id(1)

    refs_blk = refs_blk_ref[0]
    lsq_blk = jnp.float32(0.5 * _TWO_LOG2E) * jnp.sum(
        refs_blk * refs_blk, axis=0)[:, None]

    inner = jax.lax.dot_general(
        refs_blk, refs_ref[0],
        dimension_numbers=(((0,), (0,)), ((), ())),
        preferred_element_type=jnp.float32,
    )
    arg = jnp.float32(_TWO_LOG2E) * inner - lsq_ref[0] - lsq_blk
    w = jnp.exp2(arg)

    filt = jax.lax.dot_general(
        srcs_ref[0], w,
        dimension_numbers=(((1,), (1,)), ((), ())),
        preferred_element_type=jnp.float32,
    )
    out_ref[0] = filt - srcs_ref[0, :, pl.ds(i * block_n, block_n)]


def kernel(src_imgs, guide_imgs):
    bs, L, h, w = src_imgs.shape
    d = guide_imgs.shape[1]
    n = h * w

    flat_srcs = src_imgs.reshape(bs, L, n)
    refs = guide_imgs.reshape(bs, d, n)
    dp = 8
    refs_p = jnp.zeros((bs, dp, n), jnp.float32).at[:, :d, :].set(refs)
    log2e = jnp.float32(1.4426950408889634)
    lsq = log2e * jnp.sum(refs * refs, axis=1, keepdims=True)

    block_n = 512
    grid = (bs, n // block_n)

    out = pl.pallas_call(
        functools.partial(_adjacency_block, block_n=block_n),
        grid=grid,
        in_specs=[
            pl.BlockSpec((1, dp, block_n), lambda b, i: (b, 0, i)),
            pl.BlockSpec((1, dp, n), lambda b, i: (b, 0, 0)),
            pl.BlockSpec((1, 1, n), lambda b, i: (b, 0, 0)),
            pl.BlockSpec((1, L, n), lambda b, i: (b, 0, 0)),
        ],
        out_specs=pl.BlockSpec((1, L, block_n), lambda b, i: (b, 0, i)),
        out_shape=jax.ShapeDtypeStruct((bs, L, n), jnp.float32),
        compiler_params=pltpu.CompilerParams(
            dimension_semantics=("parallel", "parallel")),
    )(refs_p, refs_p, lsq, flat_srcs)

    return out.reshape(bs, L, h, w)


# speedup vs baseline: 1.0203x; 1.0203x over previous
import functools

import jax
import jax.numpy as jnp
from jax.experimental import pallas as pl
from jax.experimental.pallas import tpu as pltpu


def _adjacency_block(refs2_blk_ref, refs_ref, ssrcs_ref, e_ref, srcs_ref,
                     out_ref, *, block_n):
    i = pl.program_id(1)
    cols = pl.ds(i * block_n, block_n)

    inner2 = jax.lax.dot_general(
        refs2_blk_ref[0], refs_ref[0],
        dimension_numbers=(((0,), (0,)), ((), ())),
        preferred_element_type=jnp.float32,
    )
    w = jnp.exp(inner2)

    filt = jax.lax.dot_general(
        ssrcs_ref[0], w,
        dimension_numbers=(((1,), (1,)), ((), ())),
        preferred_element_type=jnp.float32,
    )
    out_ref[0] = filt * e_ref[0, :, cols] - srcs_ref[0, :, cols]


def kernel(src_imgs, guide_imgs):
    bs, L, h, w = src_imgs.shape
    d = guide_imgs.shape[1]
    n = h * w

    flat_srcs = src_imgs.reshape(bs, L, n)
    refs = guide_imgs.reshape(bs, d, n)
    dp = 8
    refs_p = jnp.zeros((bs, dp, n), jnp.float32).at[:, :d, :].set(refs)

    sq = jnp.sum(refs * refs, axis=1, keepdims=True)
    e = jnp.exp(-sq)
    scaled_srcs = flat_srcs * e

    block_n = 512
    grid = (bs, n // block_n)

    out = pl.pallas_call(
        functools.partial(_adjacency_block, block_n=block_n),
        grid=grid,
        in_specs=[
            pl.BlockSpec((1, dp, block_n), lambda b, i: (b, 0, i)),
            pl.BlockSpec((1, dp, n), lambda b, i: (b, 0, 0)),
            pl.BlockSpec((1, L, n), lambda b, i: (b, 0, 0)),
            pl.BlockSpec((1, 1, n), lambda b, i: (b, 0, 0)),
            pl.BlockSpec((1, L, n), lambda b, i: (b, 0, 0)),
        ],
        out_specs=pl.BlockSpec((1, L, block_n), lambda b, i: (b, 0, i)),
        out_shape=jax.ShapeDtypeStruct((bs, L, n), jnp.float32),
        compiler_params=pltpu.CompilerParams(
            dimension_semantics=("parallel", "parallel")),
    )(2.0 * refs_p, refs_p, scaled_srcs, e, flat_srcs)

    return out.reshape(bs, L, h, w)
